# baseline (speedup 1.0000x reference)
import functools

import jax
import jax.numpy as jnp
from jax.experimental import pallas as pl
from jax.experimental.pallas import tpu as pltpu
from jax.experimental.pallas import tpu_sc as plsc

N_PTS = 10000
H = 16
K_NB = 8
NPAD = 10240
CK = 5120
NC = NPAD // CK
TM = 512
GRID = (N_PTS + TM - 1) // TM
BIG = 1e30
NWORKERS = 32
B_GATHER = K_NB * NPAD
BW = B_GATHER // NWORKERS
GCHUNK = 128


def _silu(v):
    return v * jax.nn.sigmoid(v)


def _encode_body(x_ref, w1_ref, b1_ref, w2_ref, b2_ref, w3_ref, b3_ref,
                 out_ref):
    x = x_ref[:]
    h = _silu(jnp.dot(x, w1_ref[:], preferred_element_type=jnp.float32)
              + b1_ref[:])
    h = _silu(jnp.dot(h, w2_ref[:], preferred_element_type=jnp.float32)
              + b2_ref[:])
    h = jnp.dot(h, w3_ref[:], preferred_element_type=jnp.float32) + b3_ref[:]
    out_ref[:] = jnp.concatenate([h, x[:, -1:]], axis=1)


def _encode(x_pfc, w1, b1, w2, b2, w3, b3):
    n, _ = x_pfc.shape
    return pl.pallas_call(
        _encode_body,
        out_shape=jax.ShapeDtypeStruct((n, H), jnp.float32),
    )(x_pfc, w1, b1.reshape(1, -1), w2, b2.reshape(1, -1), w3,
      b3.reshape(1, -1))


def _knn_body(xq_ref, xa_ref, idx_ref, dscr, iscr):
    xq = xq_ref[:]
    sqq = jnp.sum(xq * xq, axis=1, keepdims=True)
    xq2 = xq * (-2.0)
    ones = jnp.ones((1, H), jnp.float32)
    iscr[...] = jax.lax.broadcasted_iota(
        jnp.int32, (TM, CK), 1).astype(jnp.float32)
    init = (jnp.full((TM, 1), BIG, jnp.float32),
            jnp.full((TM, 1), float(NPAD), jnp.float32))

    def merge(carry, d, c, iota):
        m, j = carry
        mc = jnp.min(d, axis=1, keepdims=True)
        jc = (c * CK) + jnp.min(jnp.where(d == mc, iota, float(CK)),
                                axis=1, keepdims=True)
        take = mc < m
        tie = mc == m
        j_new = jnp.where(take, jc, jnp.where(tie, jnp.minimum(j, jc), j))
        return jnp.minimum(m, mc), j_new

    def phase_a(c, carry):
        xk = xa_ref[c]
        prod = jax.lax.dot_general(
            xq2, xk, (((1,), (1,)), ((), ())),
            preferred_element_type=jnp.float32)
        sqk = jax.lax.dot_general(
            ones, xk * xk, (((1,), (1,)), ((), ())),
            preferred_element_type=jnp.float32)
        d = (sqq + sqk) + prod
        dscr[c] = d
        return merge(carry, d, c, iscr[...])

    m_prev, j_prev = jax.lax.fori_loop(0, NC, phase_a, init)
    picks = [j_prev]

    for t in range(K_NB - 1):
        def scan(c, carry, j_prev=j_prev, last=(t == K_NB - 2)):
            d = dscr[c]
            iota = iscr[...]
            d = jnp.where(iota == j_prev - c * CK, BIG, d)
            if not last:
                dscr[c] = d
            return merge(carry, d, c, iota)

        m_prev, j_prev = jax.lax.fori_loop(0, NC, scan, init)
        picks.append(j_prev)
    idx_ref[:] = jnp.concatenate(picks, axis=1).astype(jnp.int32)


def _knn(x_pad):
    return pl.pallas_call(
        _knn_body,
        grid=(GRID,),
        in_specs=[
            pl.BlockSpec((TM, H), lambda i: (i, 0)),
            pl.BlockSpec((NC, CK, H), lambda i: (0, 0, 0)),
        ],
        out_specs=pl.BlockSpec((TM, K_NB), lambda i: (i, 0)),
        out_shape=jax.ShapeDtypeStruct((N_PTS, K_NB), jnp.int32),
        scratch_shapes=[pltpu.VMEM((NC, TM, CK), jnp.float32),
                        pltpu.VMEM((TM, CK), jnp.float32)],
    )(x_pad, x_pad.reshape(NC, CK, H))


def _sc_gather(table, idx_flat):
    mesh = plsc.VectorSubcoreMesh(core_axis_name="c", subcore_axis_name="s")

    @functools.partial(
        pl.kernel,
        mesh=mesh,
        compiler_params=pltpu.CompilerParams(use_tc_tiling_on_sc=False),
        out_type=jax.ShapeDtypeStruct((B_GATHER, H), jnp.float32),
        scratch_types=[
            pltpu.VMEM((BW,), jnp.int32),
            pltpu.VMEM((BW, H), jnp.float32),
            pltpu.SemaphoreType.DMA,
        ],
    )
    def gk(table_hbm, idx_hbm, out_hbm, idx_v, rows_v, sem):
        wid = jax.lax.axis_index("s") * 2 + jax.lax.axis_index("c")
        base = wid * BW
        pltpu.sync_copy(idx_hbm.at[pl.ds(base, BW)], idx_v)
        for cc in range(BW // GCHUNK):
            pltpu.async_copy(
                table_hbm.at[idx_v.at[pl.ds(cc * GCHUNK, GCHUNK)]],
                rows_v.at[pl.ds(cc * GCHUNK, GCHUNK)], sem).wait()
        pltpu.sync_copy(rows_v, out_hbm.at[pl.ds(base, BW)])

    return gk(table, idx_flat)


def _post_body(xi_ref, xj_ref, cw_ref, cb_ref, w1_ref, b1_ref, w2_ref,
               b2_ref, out_ref):
    xi = xi_ref[:]
    wa = cw_ref[0:H, :]
    wb = cw_ref[H:2 * H, :]
    pre = jax.lax.dot_general(
        xi, wa, (((1,), (0,)), ((), ())),
        preferred_element_type=jnp.float32) + cb_ref[:]
    acc = jnp.zeros((TM, H), jnp.float32)
    for t in range(K_NB):
        diff = xj_ref[t] - xi
        msg = pre + jax.lax.dot_general(
            diff, wb, (((1,), (0,)), ((), ())),
            preferred_element_type=jnp.float32)
        acc = acc + _silu(msg)
    feats = acc * (1.0 / K_NB)
    h = _silu(jnp.dot(feats, w1_ref[:], preferred_element_type=jnp.float32)
              + b1_ref[:])
    out_ref[:] = (jnp.dot(h, w2_ref[:], preferred_element_type=jnp.float32)
                  + b2_ref[:])


def _post(x_pad, xj3, conv_w, conv_b, ffn_w1, ffn_b1, ffn_w2, ffn_b2):
    return pl.pallas_call(
        _post_body,
        grid=(GRID,),
        in_specs=[
            pl.BlockSpec((TM, H), lambda i: (i, 0)),
            pl.BlockSpec((K_NB, TM, H), lambda i: (0, i, 0)),
            pl.BlockSpec((2 * H, H), lambda i: (0, 0)),
            pl.BlockSpec((1, H), lambda i: (0, 0)),
            pl.BlockSpec((H, 2 * H), lambda i: (0, 0)),
            pl.BlockSpec((1, 2 * H), lambda i: (0, 0)),
            pl.BlockSpec((2 * H, H), lambda i: (0, 0)),
            pl.BlockSpec((1, H), lambda i: (0, 0)),
        ],
        out_specs=pl.BlockSpec((TM, H), lambda i: (i, 0)),
        out_shape=jax.ShapeDtypeStruct((N_PTS, H), jnp.float32),
    )(x_pad, xj3, conv_w, conv_b.reshape(1, -1), ffn_w1,
      ffn_b1.reshape(1, -1), ffn_w2, ffn_b2.reshape(1, -1))


def kernel(x_pfc, enc_w1, enc_b1, enc_w2, enc_b2, enc_w3, enc_b3, conv_w,
           conv_b, ffn_w1, ffn_b1, ffn_w2, ffn_b2):
    x_enc = _encode(x_pfc, enc_w1, enc_b1, enc_w2, enc_b2, enc_w3, enc_b3)
    x_pad = jnp.pad(x_enc, ((0, NPAD - N_PTS), (0, 0)),
                    constant_values=1e18)
    idx = _knn(x_pad)
    idx_t = jnp.pad(idx.T, ((0, 0), (0, NPAD - N_PTS)))
    xj = _sc_gather(x_enc, idx_t.reshape(-1))
    xj3 = xj.reshape(K_NB, NPAD, H)
    f = _post(x_pad, xj3, conv_w, conv_b, ffn_w1, ffn_b1, ffn_w2, ffn_b2)
    return jnp.concatenate([f, x_pfc], axis=1)


# speedup vs baseline: 1.0369x; 1.0369x over previous
import functools

import jax
import jax.numpy as jnp
from jax.experimental import pallas as pl
from jax.experimental.pallas import tpu as pltpu
from jax.experimental.pallas import tpu_sc as plsc

N_PTS = 10000
H = 16
K_NB = 8
NPAD = 10240
CK = 2048
NC = NPAD // CK
TM = 640
GRID = (N_PTS + TM - 1) // TM
BIG = 1e30
NWORKERS = 32
B_GATHER = K_NB * NPAD
BW = B_GATHER // NWORKERS
GCHUNK = 128


def _silu(v):
    return v * jax.nn.sigmoid(v)


def _encode_body(x_ref, w1_ref, b1_ref, w2_ref, b2_ref, w3_ref, b3_ref,
                 out_ref):
    x = x_ref[:]
    h = _silu(jnp.dot(x, w1_ref[:], preferred_element_type=jnp.float32)
              + b1_ref[:])
    h = _silu(jnp.dot(h, w2_ref[:], preferred_element_type=jnp.float32)
              + b2_ref[:])
    h = jnp.dot(h, w3_ref[:], preferred_element_type=jnp.float32) + b3_ref[:]
    out_ref[:] = jnp.concatenate([h, x[:, -1:]], axis=1)


def _encode(x_pfc, w1, b1, w2, b2, w3, b3):
    n, _ = x_pfc.shape
    return pl.pallas_call(
        _encode_body,
        out_shape=jax.ShapeDtypeStruct((n, H), jnp.float32),
    )(x_pfc, w1, b1.reshape(1, -1), w2, b2.reshape(1, -1), w3,
      b3.reshape(1, -1))


def _knn_body(xq_ref, xa_ref, idx_ref, dscr, iscr):
    xq = xq_ref[:]
    sqq = jnp.sum(xq * xq, axis=1, keepdims=True)
    xq2 = xq * (-2.0)
    ones = jnp.ones((1, H), jnp.float32)
    iscr[...] = jax.lax.broadcasted_iota(
        jnp.int32, (TM, CK), 1).astype(jnp.float32)
    init = (jnp.full((TM, 1), BIG, jnp.float32),
            jnp.full((TM, 1), float(NPAD), jnp.float32))

    def merge(carry, d, c, iota):
        m, j = carry
        mc = jnp.min(d, axis=1, keepdims=True)
        jc = (c * CK) + jnp.min(jnp.where(d == mc, iota, float(CK)),
                                axis=1, keepdims=True)
        take = mc < m
        tie = mc == m
        j_new = jnp.where(take, jc, jnp.where(tie, jnp.minimum(j, jc), j))
        return jnp.minimum(m, mc), j_new

    def phase_a(c, carry):
        xk = xa_ref[c]
        prod = jax.lax.dot_general(
            xq2, xk, (((1,), (1,)), ((), ())),
            preferred_element_type=jnp.float32)
        sqk = jax.lax.dot_general(
            ones, xk * xk, (((1,), (1,)), ((), ())),
            preferred_element_type=jnp.float32)
        d = (sqq + sqk) + prod
        dscr[c] = d
        return merge(carry, d, c, iscr[...])

    m_prev, j_prev = jax.lax.fori_loop(0, NC, phase_a, init)
    picks = [j_prev]

    for t in range(K_NB - 1):
        def scan(c, carry, j_prev=j_prev, last=(t == K_NB - 2)):
            d = dscr[c]
            iota = iscr[...]
            d = jnp.where(iota == j_prev - c * CK, BIG, d)
            if not last:
                dscr[c] = d
            return merge(carry, d, c, iota)

        m_prev, j_prev = jax.lax.fori_loop(0, NC, scan, init)
        picks.append(j_prev)
    idx_ref[:] = jnp.concatenate(picks, axis=1).astype(jnp.int32)


def _knn(x_pad):
    return pl.pallas_call(
        _knn_body,
        grid=(GRID,),
        in_specs=[
            pl.BlockSpec((TM, H), lambda i: (i, 0)),
            pl.BlockSpec((NC, CK, H), lambda i: (0, 0, 0)),
        ],
        out_specs=pl.BlockSpec((TM, K_NB), lambda i: (i, 0)),
        out_shape=jax.ShapeDtypeStruct((N_PTS, K_NB), jnp.int32),
        scratch_shapes=[pltpu.VMEM((NC, TM, CK), jnp.float32),
                        pltpu.VMEM((TM, CK), jnp.float32)],
    )(x_pad, x_pad.reshape(NC, CK, H))


def _sc_gather(table, idx_flat):
    mesh = plsc.VectorSubcoreMesh(core_axis_name="c", subcore_axis_name="s")

    @functools.partial(
        pl.kernel,
        mesh=mesh,
        compiler_params=pltpu.CompilerParams(use_tc_tiling_on_sc=False),
        out_type=jax.ShapeDtypeStruct((B_GATHER, H), jnp.float32),
        scratch_types=[
            pltpu.VMEM((BW,), jnp.int32),
            pltpu.VMEM((BW, H), jnp.float32),
            pltpu.SemaphoreType.DMA,
        ],
    )
    def gk(table_hbm, idx_hbm, out_hbm, idx_v, rows_v, sem):
        wid = jax.lax.axis_index("s") * 2 + jax.lax.axis_index("c")
        base = wid * BW
        pltpu.sync_copy(idx_hbm.at[pl.ds(base, BW)], idx_v)
        for cc in range(BW // GCHUNK):
            pltpu.async_copy(
                table_hbm.at[idx_v.at[pl.ds(cc * GCHUNK, GCHUNK)]],
                rows_v.at[pl.ds(cc * GCHUNK, GCHUNK)], sem).wait()
        pltpu.sync_copy(rows_v, out_hbm.at[pl.ds(base, BW)])

    return gk(table, idx_flat)


def _post_body(xi_ref, xj_ref, cw_ref, cb_ref, w1_ref, b1_ref, w2_ref,
               b2_ref, out_ref):
    xi = xi_ref[:]
    wa = cw_ref[0:H, :]
    wb = cw_ref[H:2 * H, :]
    pre = jax.lax.dot_general(
        xi, wa, (((1,), (0,)), ((), ())),
        preferred_element_type=jnp.float32) + cb_ref[:]
    acc = jnp.zeros((TM, H), jnp.float32)
    for t in range(K_NB):
        diff = xj_ref[t] - xi
        msg = pre + jax.lax.dot_general(
            diff, wb, (((1,), (0,)), ((), ())),
            preferred_element_type=jnp.float32)
        acc = acc + _silu(msg)
    feats = acc * (1.0 / K_NB)
    h = _silu(jnp.dot(feats, w1_ref[:], preferred_element_type=jnp.float32)
              + b1_ref[:])
    out_ref[:] = (jnp.dot(h, w2_ref[:], preferred_element_type=jnp.float32)
                  + b2_ref[:])


def _post(x_pad, xj3, conv_w, conv_b, ffn_w1, ffn_b1, ffn_w2, ffn_b2):
    return pl.pallas_call(
        _post_body,
        grid=(GRID,),
        in_specs=[
            pl.BlockSpec((TM, H), lambda i: (i, 0)),
            pl.BlockSpec((K_NB, TM, H), lambda i: (0, i, 0)),
            pl.BlockSpec((2 * H, H), lambda i: (0, 0)),
            pl.BlockSpec((1, H), lambda i: (0, 0)),
            pl.BlockSpec((H, 2 * H), lambda i: (0, 0)),
            pl.BlockSpec((1, 2 * H), lambda i: (0, 0)),
            pl.BlockSpec((2 * H, H), lambda i: (0, 0)),
            pl.BlockSpec((1, H), lambda i: (0, 0)),
        ],
        out_specs=pl.BlockSpec((TM, H), lambda i: (i, 0)),
        out_shape=jax.ShapeDtypeStruct((N_PTS, H), jnp.float32),
    )(x_pad, xj3, conv_w, conv_b.reshape(1, -1), ffn_w1,
      ffn_b1.reshape(1, -1), ffn_w2, ffn_b2.reshape(1, -1))


def kernel(x_pfc, enc_w1, enc_b1, enc_w2, enc_b2, enc_w3, enc_b3, conv_w,
           conv_b, ffn_w1, ffn_b1, ffn_w2, ffn_b2):
    x_enc = _encode(x_pfc, enc_w1, enc_b1, enc_w2, enc_b2, enc_w3, enc_b3)
    x_pad = jnp.pad(x_enc, ((0, NPAD - N_PTS), (0, 0)),
                    constant_values=1e18)
    idx = _knn(x_pad)
    idx_t = jnp.pad(idx.T, ((0, 0), (0, NPAD - N_PTS)))
    xj = _sc_gather(x_enc, idx_t.reshape(-1))
    xj3 = xj.reshape(K_NB, NPAD, H)
    f = _post(x_pad, xj3, conv_w, conv_b, ffn_w1, ffn_b1, ffn_w2, ffn_b2)
    return jnp.concatenate([f, x_pfc], axis=1)
